# Initial kernel scaffold; baseline (speedup 1.0000x reference)
import jax
import jax.numpy as jnp
from jax.experimental import pallas as pl


def kernel(in_tensor, indices):
    raise NotImplementedError("write your pallas kernel here")


# speedup vs baseline: 1.4778x; 1.4778x over previous
import functools

import jax
import jax.numpy as jnp
from jax import lax
from jax.experimental import pallas as pl
from jax.experimental.pallas import tpu as pltpu
from jax.experimental.pallas import tpu_sc as plsc

_B, _H, _W, _C = 4, 224, 224, 384
_N = 16384
_S = _B * _N
_NC, _NS = 2, 16
_NW = _NC * _NS
_SPW = _S // _NW
_K = 16
_NCHUNK = _SPW // _K
_L = 16
_CCH = _C // _L
_UNROLL = 8


def _body(ys_hbm, xs_hbm, tbl_hbm, out_hbm,
          ysv, xsv, idxv, gv, ov, sg0, sg1, so0, so1):
    wid = lax.axis_index("s") * _NC + lax.axis_index("c")
    base = wid * _SPW
    bofs = (wid // (_NW // _B)) * (_H * _W)

    pltpu.sync_copy(ys_hbm.at[pl.ds(base, _SPW)], ysv)
    pltpu.sync_copy(xs_hbm.at[pl.ds(base, _SPW)], xsv)

    gsems = (sg0, sg1)
    osems = (so0, so1)

    def issue(ci, slot):
        y = ysv[pl.ds(ci * _K, _K)]
        x = xsv[pl.ds(ci * _K, _K)]
        iy = y.astype(jnp.int32)
        ix = x.astype(jnp.int32)
        flat = bofs + iy * _W + ix
        idxv[slot, pl.ds(0, _K)] = flat
        idxv[slot, pl.ds(_K, _K)] = flat + 1
        idxv[slot, pl.ds(2 * _K, _K)] = flat + _W
        idxv[slot, pl.ds(3 * _K, _K)] = flat + _W + 1
        pltpu.async_copy(tbl_hbm.at[idxv.at[slot]], gv.at[slot], gsems[slot])

    def wait_gather(slot):
        pltpu.make_async_copy(
            tbl_hbm.at[idxv.at[slot]], gv.at[slot], gsems[slot]).wait()

    def wait_out(slot):
        pltpu.make_async_copy(
            ov.at[slot], out_hbm.at[pl.ds(0, _K)], osems[slot]).wait()

    def blend(ci, slot):
        y = ysv[pl.ds(ci * _K, _K)]
        x = xsv[pl.ds(ci * _K, _K)]
        fy_all = y - y.astype(jnp.int32).astype(jnp.float32)
        fx_all = x - x.astype(jnp.int32).astype(jnp.float32)
        for s in range(_K):
            fy = jnp.full((_L,), fy_all[s], jnp.float32)
            fx = jnp.full((_L,), fx_all[s], jnp.float32)

            def step(u, carry):
                for j in range(_UNROLL):
                    sl = pl.ds((u * _UNROLL + j) * _L, _L)
                    g00 = gv[slot, s, sl]
                    g01 = gv[slot, _K + s, sl]
                    g10 = gv[slot, 2 * _K + s, sl]
                    g11 = gv[slot, 3 * _K + s, sl]
                    top = g00 + fx * (g01 - g00)
                    bot = g10 + fx * (g11 - g10)
                    ov[slot, s, sl] = top + fy * (bot - top)
                return carry

            lax.fori_loop(0, _CCH // _UNROLL, step, 0)
        pltpu.async_copy(
            ov.at[slot], out_hbm.at[pl.ds(base + ci * _K, _K)], osems[slot])

    issue(0, 0)

    def outer(i, carry):
        t = i * 2
        issue(t + 1, 1)
        wait_gather(0)

        @pl.when(i > 0)
        def _():
            wait_out(0)

        blend(t, 0)

        @pl.when(t + 2 < _NCHUNK)
        def _():
            issue(t + 2, 0)

        wait_gather(1)

        @pl.when(i > 0)
        def _():
            wait_out(1)

        blend(t + 1, 1)
        return carry

    lax.fori_loop(0, _NCHUNK // 2, outer, 0)
    wait_out(0)
    wait_out(1)


_grid_sample = functools.partial(
    pl.kernel,
    mesh=plsc.VectorSubcoreMesh(core_axis_name="c", subcore_axis_name="s"),
    out_type=jax.ShapeDtypeStruct((_S, _C), jnp.float32),
    scratch_types=[
        pltpu.VMEM((_SPW,), jnp.float32),
        pltpu.VMEM((_SPW,), jnp.float32),
        pltpu.VMEM((2, 4 * _K), jnp.int32),
        pltpu.VMEM((2, 4 * _K, _C), jnp.float32),
        pltpu.VMEM((2, _K, _C), jnp.float32),
        pltpu.SemaphoreType.DMA,
        pltpu.SemaphoreType.DMA,
        pltpu.SemaphoreType.DMA,
        pltpu.SemaphoreType.DMA,
    ],
)(_body)


def kernel(in_tensor, indices):
    tbl = in_tensor.reshape(_B * _H * _W, _C)
    ys = indices[..., 0].reshape(_S)
    xs = indices[..., 1].reshape(_S)
    out = _grid_sample(ys, xs, tbl)
    return out.reshape(_B, _N, _C)
